# Initial kernel scaffold; baseline (speedup 1.0000x reference)
import jax
import jax.numpy as jnp
from jax.experimental import pallas as pl


def kernel(x, edge_index, emb, W_z, b_z, W_r, b_r, W_h, b_h, lz_W, lz_b, lr_W, lr_b, lh_W, lh_b, lin_W, lin_b):
    raise NotImplementedError("write your pallas kernel here")


# speedup vs baseline: 166.8104x; 166.8104x over previous
import functools

import jax
import jax.numpy as jnp
from jax import lax
from jax.experimental import pallas as pl
from jax.experimental.pallas import tpu as pltpu
from jax.experimental.pallas import tpu_sc as plsc

B = 4
N = 10000
S = 12
E = 320000
NC_CLS = 10
PERIODS = 4
HID = 32

NUM_TILES = 32
EPT_DEG = E // NUM_TILES
NCHUNK = NUM_TILES // B
EPT_HIST = E // NCHUNK
EBLK = 2000
HISTN = N * NC_CLS

_mesh = plsc.VectorSubcoreMesh(core_axis_name="c", subcore_axis_name="s")
_sc_params = pltpu.CompilerParams(needs_layout_passes=False)


@functools.partial(
    pl.kernel,
    mesh=_mesh,
    out_type=jax.ShapeDtypeStruct((NUM_TILES * N,), jnp.float32),
    scratch_types=[
        pltpu.VMEM((N,), jnp.float32),
        pltpu.VMEM((EBLK,), jnp.int32),
    ],
    compiler_params=_sc_params,
)
def _deg_kernel(dst_hbm, zeros_hbm, out_hbm, deg_v, dbuf_v):
    wid = lax.axis_index("s") * 2 + lax.axis_index("c")
    pltpu.sync_copy(zeros_hbm.at[pl.ds(0, N)], deg_v)
    ones = jnp.ones((16,), jnp.float32)
    ebase = wid * EPT_DEG

    def blk_body(blk, carry):
        pltpu.sync_copy(dst_hbm.at[pl.ds(ebase + blk * EBLK, EBLK)], dbuf_v)

        def step(k, c2):
            d16 = dbuf_v[pl.ds(k * 16, 16)]
            plsc.addupdate_scatter(deg_v, [d16], ones)
            return c2

        return lax.fori_loop(0, EBLK // 16, step, carry)

    lax.fori_loop(0, EPT_DEG // EBLK, blk_body, 0)
    pltpu.sync_copy(deg_v, out_hbm.at[pl.ds(wid * N, N)])


def _norm_body(pd_ref, dis_ref, inv_ref):
    deg = jnp.sum(pd_ref[...], axis=0) + 1.0
    dis_ref[...] = lax.rsqrt(deg)
    inv_ref[...] = 1.0 / deg


def _norm_call(pdeg):
    return pl.pallas_call(
        _norm_body,
        out_shape=(
            jax.ShapeDtypeStruct((N,), jnp.float32),
            jax.ShapeDtypeStruct((N,), jnp.float32),
        ),
    )(pdeg)


@functools.partial(
    pl.kernel,
    mesh=_mesh,
    out_type=jax.ShapeDtypeStruct((NUM_TILES * HISTN,), jnp.float32),
    scratch_types=[
        pltpu.VMEM((HISTN,), jnp.float32),
        pltpu.VMEM((N,), jnp.float32),
        pltpu.VMEM((N,), jnp.int32),
        pltpu.VMEM((EBLK,), jnp.int32),
        pltpu.VMEM((EBLK,), jnp.int32),
    ],
    compiler_params=_sc_params,
)
def _hist_kernel(src_hbm, dst_hbm, cls_hbm, dis_hbm, zeros_hbm, out_hbm,
                 hist_v, dis_v, cls_v, sbuf_v, dbuf_v):
    wid = lax.axis_index("s") * 2 + lax.axis_index("c")
    b = lax.rem(wid, B)
    j = lax.div(wid, B)
    pltpu.sync_copy(zeros_hbm.at[pl.ds(0, HISTN)], hist_v)
    pltpu.sync_copy(dis_hbm.at[pl.ds(0, N)], dis_v)
    pltpu.sync_copy(cls_hbm.at[pl.ds(b * N, N)], cls_v)
    ebase = j * EPT_HIST
    ten = jnp.full((16,), 10, jnp.int32)

    def blk_body(blk, carry):
        base = ebase + blk * EBLK
        pltpu.sync_copy(src_hbm.at[pl.ds(base, EBLK)], sbuf_v)
        pltpu.sync_copy(dst_hbm.at[pl.ds(base, EBLK)], dbuf_v)

        def step(k, c2):
            s16 = sbuf_v[pl.ds(k * 16, 16)]
            d16 = dbuf_v[pl.ds(k * 16, 16)]
            val = plsc.load_gather(dis_v, [s16])
            c16 = plsc.load_gather(cls_v, [s16])
            idx = d16 * ten + c16
            plsc.addupdate_scatter(hist_v, [idx], val)
            return c2

        return lax.fori_loop(0, EBLK // 16, step, carry)

    lax.fori_loop(0, EPT_HIST // EBLK, blk_body, 0)
    pltpu.sync_copy(hist_v, out_hbm.at[pl.ds(wid * HISTN, HISTN)])


_BN = 2000
_NB_PER_BATCH = N // _BN


def _head_body(ph_ref, cls_ref, dis_ref, inv_ref, emb_ref, wz_ref, bz_ref,
               wh_ref, bh_ref, lzw_ref, lzb_ref, lhw_ref, lhb_ref,
               linw_ref, linb_ref, out_ref):
    ph = ph_ref[...]
    h = jnp.sum(ph.reshape(NCHUNK, _BN, NC_CLS), axis=0)
    h = h * dis_ref[...].reshape(_BN)[:, None]
    cls = cls_ref[...].reshape(_BN)
    onehot = lax.broadcasted_iota(jnp.int32, (_BN, NC_CLS), 1) == cls[:, None]
    h = h + jnp.where(onehot, inv_ref[...].reshape(_BN)[:, None], 0.0)

    lzw1 = lzw_ref[...][:HID, :]
    lhw1 = lhw_ref[...][:HID, :]
    gz = emb_ref[...] @ wz_ref[...] @ lzw1
    gh = emb_ref[...] @ wh_ref[...] @ lhw1
    cz = bz_ref[...][None, :] @ lzw1 + lzb_ref[...][None, :]
    ch = bh_ref[...][None, :] @ lhw1 + lhb_ref[...][None, :]

    z = jax.nn.sigmoid(h @ gz + cz)
    ht = jnp.tanh(h @ gh + ch)
    g = jax.nn.relu((1.0 - z) * ht)
    y = g @ linw_ref[...] + linb_ref[...][None, :]
    y4 = y.reshape(_BN, PERIODS, NC_CLS)
    m = jnp.max(y4, axis=-1, keepdims=True)
    e = jnp.exp(y4 - m)
    sm = e / jnp.sum(e, axis=-1, keepdims=True)
    out_ref[...] = sm.reshape(_BN, PERIODS * NC_CLS)


def _head_call(ph, cls_flat, dis, inv, emb, W_z, b_z, W_h, b_h,
               lz_W, lz_b, lh_W, lh_b, lin_W, lin_b):
    nsteps = (B * N) // _BN
    full = lambda shape: pl.BlockSpec(shape, lambda i: tuple(0 for _ in shape))
    return pl.pallas_call(
        _head_body,
        grid=(nsteps,),
        in_specs=[
            pl.BlockSpec((NCHUNK, 1, _BN, NC_CLS), lambda i: (0, i, 0, 0)),
            pl.BlockSpec((1, 1, _BN), lambda i: (i, 0, 0)),
            pl.BlockSpec((1, 1, _BN), lambda i: (i % _NB_PER_BATCH, 0, 0)),
            pl.BlockSpec((1, 1, _BN), lambda i: (i % _NB_PER_BATCH, 0, 0)),
            full((NC_CLS, NC_CLS)),
            full((NC_CLS, HID)),
            full((HID,)),
            full((NC_CLS, HID)),
            full((HID,)),
            full((2 * HID, HID)),
            full((HID,)),
            full((2 * HID, HID)),
            full((HID,)),
            full((HID, PERIODS * NC_CLS)),
            full((PERIODS * NC_CLS,)),
        ],
        out_specs=pl.BlockSpec((_BN, PERIODS * NC_CLS), lambda i: (i, 0)),
        out_shape=jax.ShapeDtypeStruct((B * N, PERIODS * NC_CLS), jnp.float32),
    )(ph, cls_flat, dis, inv, emb, W_z, b_z, W_h, b_h,
      lz_W, lz_b, lh_W, lh_b, lin_W, lin_b)


def kernel(x, edge_index, emb, W_z, b_z, W_r, b_r, W_h, b_h,
           lz_W, lz_b, lr_W, lr_b, lh_W, lh_b, lin_W, lin_b):
    del W_r, b_r, lr_W, lr_b
    src = edge_index[0]
    dst = edge_index[1]
    cls_flat = x[:, S - 1].astype(jnp.int32)
    zeros = jnp.zeros((HISTN,), jnp.float32)

    pdeg = _deg_kernel(dst, zeros)
    dis, inv = _norm_call(pdeg.reshape(NUM_TILES, N))
    ph = _hist_kernel(src, dst, cls_flat, dis, zeros)
    nsteps = (B * N) // _BN
    out = _head_call(ph.reshape(NCHUNK, nsteps, _BN, NC_CLS),
                     cls_flat.reshape(nsteps, 1, _BN),
                     dis.reshape(_NB_PER_BATCH, 1, _BN),
                     inv.reshape(_NB_PER_BATCH, 1, _BN),
                     emb, W_z, b_z, W_h, b_h,
                     lz_W, lz_b, lh_W, lh_b, lin_W, lin_b)
    return out.reshape(B * N, PERIODS, NC_CLS)
